# baseline (speedup 1.0000x reference)
import functools

import jax
import jax.numpy as jnp
from jax import lax
from jax.experimental import pallas as pl
from jax.experimental.pallas import tpu as pltpu
from jax.experimental.pallas import tpu_sc as plsc

VOCAB = 8192
B, T = 8, 512
N = B * T
NW = 32
NSUB = 16
ROWS_PER_W = N // NW

HA = 64
NSUPER = HA // 16
NBUF_A = 4

HB = ROWS_PER_W - HA
KB = 2
NCHUNK_B = HB // KB
NBUF_B = 4

_mesh = plsc.VectorSubcoreMesh(core_axis_name="c", subcore_axis_name="s")


@functools.partial(
    pl.kernel,
    out_type=jax.ShapeDtypeStruct((N, VOCAB), jnp.float32),
    mesh=_mesh,
    scratch_types=[
        pltpu.VMEM((HA,), jnp.int32),
        pltpu.VMEM((NCHUNK_B, KB), jnp.int32),
        pltpu.MemorySpace.VMEM_SHARED((NSUB, NBUF_A, 1, VOCAB), jnp.float32),
        pltpu.VMEM((NBUF_B, KB, VOCAB), jnp.float32),
        pltpu.SemaphoreType.DMA((NBUF_A,)),
        pltpu.SemaphoreType.DMA((NBUF_A,)),
        pltpu.SemaphoreType.DMA((NBUF_B,)),
        pltpu.SemaphoreType.DMA((NBUF_B,)),
    ],
)
def _gather_sc(idx_hbm, idx2_hbm, emb_hbm, out_hbm,
               idx_a, idx_b, rows_sh, rows_v, gsa, ssa, gsb, ssb):
    sid = lax.axis_index("s")
    wid = sid * 2 + lax.axis_index("c")
    base = wid * ROWS_PER_W
    base_b = base + HA
    pltpu.sync_copy(idx_hbm.at[pl.ds(base, HA)], idx_a)
    idx2_off = wid * (ROWS_PER_W // KB) + HA // KB
    pltpu.sync_copy(idx2_hbm.at[pl.ds(idx2_off, NCHUNK_B)], idx_b)

    def gather_a(row, b):
        pltpu.async_copy(emb_hbm.at[pl.ds(row, 1)], rows_sh.at[sid, b],
                         gsa.at[b])

    def store_a(c, b):
        pltpu.async_copy(rows_sh.at[sid, b],
                         out_hbm.at[pl.ds(base + c, 1)], ssa.at[b])

    def wait_ga(b):
        pltpu.make_async_copy(emb_hbm.at[pl.ds(0, 1)], rows_sh.at[sid, b],
                              gsa.at[b]).wait()

    def wait_sa(b):
        pltpu.make_async_copy(rows_sh.at[sid, b], out_hbm.at[pl.ds(base, 1)],
                              ssa.at[b]).wait()

    def idx_vec(g):
        return idx_a[pl.ds(g * 16, 16)]

    def a_phase(c0, drain_only=False, vec=None, lanes=None):
        for j in range(NBUF_A):
            wait_ga(j)
            store_a(c0 + j, j)
        if not drain_only:
            for j in range(NBUF_A):
                wait_sa(j)
                gather_a(vec[lanes + j], j)

    def gather_b(c, b):
        pltpu.async_copy(emb_hbm.at[idx_b.at[c]], rows_v.at[b], gsb.at[b])

    def store_b(c, b):
        pltpu.async_copy(rows_v.at[b],
                         out_hbm.at[pl.ds(base_b + c * KB, KB)], ssb.at[b])

    def wait_gb(b):
        pltpu.make_async_copy(emb_hbm.at[pl.ds(0, KB)], rows_v.at[b],
                              gsb.at[b]).wait()

    def wait_sb(b):
        pltpu.make_async_copy(rows_v.at[b], out_hbm.at[pl.ds(base_b, KB)],
                              ssb.at[b]).wait()

    def b_block(k, last=False):
        for b in range(NBUF_B):
            wait_gb(b)
            store_b(k * NBUF_B + b, b)
        if not last:
            for b in range(NBUF_B):
                wait_sb(b)
                gather_b((k + 1) * NBUF_B + b, b)

    v0 = idx_vec(0)
    for j in range(NBUF_A):
        gather_a(v0[j], j)
    for b in range(NBUF_B):
        gather_b(b, b)

    def body(g, carry):
        vec = idx_vec(g)
        nxt = idx_vec(g + 1)
        c0 = g * 16
        a_phase(c0, vec=vec, lanes=4)
        a_phase(c0 + 4, vec=vec, lanes=8)
        b_block(2 * g)
        a_phase(c0 + 8, vec=vec, lanes=12)
        a_phase(c0 + 12, vec=nxt, lanes=0)
        b_block(2 * g + 1)
        return carry

    lax.fori_loop(0, NSUPER - 1, body, 0)

    vec = idx_vec(NSUPER - 1)
    c0 = (NSUPER - 1) * 16
    a_phase(c0, vec=vec, lanes=4)
    a_phase(c0 + 4, vec=vec, lanes=8)
    b_block(2 * (NSUPER - 1))
    a_phase(c0 + 8, vec=vec, lanes=12)
    a_phase(c0 + 12, drain_only=True)
    b_block(2 * (NSUPER - 1) + 1, last=True)
    for j in range(NBUF_A):
        wait_sa(j)
    for b in range(NBUF_B):
        wait_sb(b)


def kernel(x, emb):
    idx_flat = x.reshape(N)
    idx2 = x.reshape(N // KB, KB)
    out = _gather_sc(idx_flat, idx2, emb)
    return out.reshape(B, T, VOCAB)


# speedup vs baseline: 1.0685x; 1.0685x over previous
import functools

import jax
import jax.numpy as jnp
from jax import lax
from jax.experimental import pallas as pl
from jax.experimental.pallas import tpu as pltpu
from jax.experimental.pallas import tpu_sc as plsc

VOCAB = 8192
N = 4096
NW = 32
NSUB = 16
ROWS_PER_W = N // NW
NBUF = 8
NGRP = ROWS_PER_W // 16

_mesh = plsc.VectorSubcoreMesh(core_axis_name="c", subcore_axis_name="s")


@functools.partial(
    pl.kernel,
    out_type=jax.ShapeDtypeStruct((N, VOCAB), jnp.float32),
    mesh=_mesh,
    scratch_types=[
        pltpu.VMEM((ROWS_PER_W,), jnp.int32),
        pltpu.MemorySpace.VMEM_SHARED((NSUB, NBUF, 1, VOCAB), jnp.float32),
        pltpu.SemaphoreType.DMA((NBUF,)),
        pltpu.SemaphoreType.DMA((NBUF,)),
    ],
)
def _gather_sc(idx_hbm, emb_hbm, out_hbm, idx_v, rows_sh, gsem, ssem):
    sid = lax.axis_index("s")
    wid = sid * 2 + lax.axis_index("c")
    base = wid * ROWS_PER_W
    pltpu.sync_copy(idx_hbm.at[pl.ds(base, ROWS_PER_W)], idx_v)

    def gather(row, b):
        pltpu.async_copy(emb_hbm.at[pl.ds(row, 1)], rows_sh.at[sid, b],
                         gsem.at[b])

    def store(c, b):
        pltpu.async_copy(rows_sh.at[sid, b],
                         out_hbm.at[pl.ds(base + c, 1)], ssem.at[b])

    def wait_g(b):
        pltpu.make_async_copy(emb_hbm.at[pl.ds(0, 1)], rows_sh.at[sid, b],
                              gsem.at[b]).wait()

    def wait_s(b):
        pltpu.make_async_copy(rows_sh.at[sid, b], out_hbm.at[pl.ds(base, 1)],
                              ssem.at[b]).wait()

    def idx_vec(g):
        return idx_v[pl.ds(g * 16, 16)]

    v0 = idx_vec(0)
    for j in range(NBUF):
        gather(v0[j], j)

    def body(g, carry):
        vec = idx_vec(g)
        nxt = idx_vec(g + 1)
        c0 = g * 16
        for j in range(NBUF):
            wait_g(j)
            store(c0 + j, j)
        for j in range(NBUF):
            wait_s(j)
            gather(vec[8 + j], j)
        for j in range(NBUF):
            wait_g(j)
            store(c0 + 8 + j, j)
        for j in range(NBUF):
            wait_s(j)
            gather(nxt[j], j)
        return carry

    lax.fori_loop(0, NGRP - 1, body, 0)

    vec = idx_vec(NGRP - 1)
    c0 = (NGRP - 1) * 16
    for j in range(NBUF):
        wait_g(j)
        store(c0 + j, j)
    for j in range(NBUF):
        wait_s(j)
        gather(vec[8 + j], j)
    for j in range(NBUF):
        wait_g(j)
        store(c0 + 8 + j, j)
    for j in range(NBUF):
        wait_s(j)


def kernel(x, emb):
    out = _gather_sc(x.reshape(N), emb)
    return out.reshape(8, 512, VOCAB)
